# baseline (speedup 1.0000x reference)
import jax
import jax.numpy as jnp
from jax.experimental import pallas as pl
from jax.experimental.pallas import tpu as pltpu

_N = 9216
_D = 384
_NB = 4
_TH = 0.01
_NG = _N // 8
_GPC = 165


def _rownorm(c):
    n2 = jnp.sum(c * c, axis=1, keepdims=True)
    return c / jnp.maximum(jnp.sqrt(n2), 1e-12)


def _tree8(acc):
    b = jnp.concatenate([acc[4:8], acc[0:4]], axis=0) + acc
    c = jnp.concatenate([b[2:8], b[0:2]], axis=0) + b
    d = jnp.concatenate([c[1:8], c[0:1]], axis=0) + c
    return d[0:1]


def _masked_sums(fc_ref, m_ref):
    S0 = jnp.zeros((1, _D), jnp.float32)
    S1 = jnp.zeros((1, _D), jnp.float32)
    for c in range(-(-_NG // _GPC)):
        glo = c * _GPC
        gn = min(_GPC, _NG - glo)

        def body(g, accs):
            a0, a1 = accs
            base = (glo + g) * 8
            f8 = fc_ref[pl.ds(base, 8), :]
            m8 = m_ref[pl.ds(base, 8), :]
            return (f8 * (1.0 - m8) + a0, f8 * m8 + a1)

        a0, a1 = jax.lax.fori_loop(
            0, gn, body,
            (jnp.zeros((8, _D), jnp.float32), jnp.zeros((8, _D), jnp.float32)))
        S0 = S0 + _tree8(a0)
        S1 = S1 + _tree8(a1)
    return jnp.concatenate([S0, S1], axis=0)


def _body(fc_ref, fnb_ref, c_ref, cout_ref, lab_ref, oh0_ref, oh1_ref,
          w_ref, cini_ref, m_ref):
    Fnb = fnb_ref[...]

    def step(centers):
        nc = _rownorm(centers)
        ncb = nc.astype(jnp.bfloat16)
        s = jax.lax.dot_general(
            ncb, Fnb, (((1,), (0,)), ((), ())),
            preferred_element_type=jnp.float32)
        d = 0.5 * (1.0 - s)
        lab = (d[1:2] < d[0:1]).astype(jnp.float32)
        m_ref[...] = lab.T
        S = _masked_sums(fc_ref, m_ref)
        cnt1 = jnp.sum(lab)
        cnts = jnp.concatenate(
            [jnp.reshape(_N - cnt1 + 1.0, (1, 1)),
             jnp.reshape(cnt1 + 1.0, (1, 1))], axis=0)
        cI = S / cnts
        Cd = jnp.sum(_rownorm(cI) * nc) * 0.5
        return d, lab, cI, Cd

    def cond_step(state):
        d, lab, cI, cdist, done = state
        d2, lb2, cI2, Cd2 = step(cI)
        d = jnp.where(done, d, d2)
        lab = jnp.where(done, lab, lb2)
        cI = jnp.where(done, cI, cI2)
        cdist = jnp.where(done, cdist, Cd2)
        done = jnp.logical_or(done, cdist < _TH)
        return (d, lab, cI, cdist, done), Cd2

    def record(b, state):
        d, lab, cI, _, _ = state
        lab_ref[b:b + 1, :] = lab.astype(jnp.int32)
        oh0_ref[b:b + 1, :] = 1.0 - lab
        oh1_ref[b:b + 1, :] = lab
        dmax = jnp.max(d, axis=1, keepdims=True)
        dmin = jnp.min(d, axis=1, keepdims=True)
        dn = -d / (dmax - dmin + 1e-7)
        w = 1.0 - dn + 0.1
        w_ref[b:b + 1, :] = w[0:1] * (1.0 - lab) + w[1:2] * lab

    d, lab, cI, Cd = step(c_ref[...])
    cini = Cd
    state = (d, lab, cI, Cd, Cd < _TH)
    for _ in range(3):
        state, _ = cond_step(state)
    record(0, state)
    csum = state[2]
    for b in range(1, _NB):
        done_prev = state[4]
        state, Cd2 = cond_step(state)
        cini = cini + jnp.where(done_prev, 0.0, Cd2)
        for _ in range(3):
            state, _ = cond_step(state)
        record(b, state)
        csum = csum + state[2]
    cout_ref[...] = csum / _NB
    cini_ref[...] = jnp.reshape(cini / _NB, (1, 1))


def kernel(FeatureT, centerInit):
    Fb = FeatureT[0].reshape(_D, _N).T
    n = jnp.linalg.norm(Fb, axis=1, keepdims=True)
    Fnb = (Fb / jnp.maximum(n, 1e-12)).astype(jnp.bfloat16).T
    outs = pl.pallas_call(
        _body,
        out_shape=(
            jax.ShapeDtypeStruct((2, _D), jnp.float32),
            jax.ShapeDtypeStruct((_NB, _N), jnp.int32),
            jax.ShapeDtypeStruct((_NB, _N), jnp.float32),
            jax.ShapeDtypeStruct((_NB, _N), jnp.float32),
            jax.ShapeDtypeStruct((_NB, _N), jnp.float32),
            jax.ShapeDtypeStruct((1, 1), jnp.float32),
        ),
        scratch_shapes=[pltpu.VMEM((_N, 1), jnp.float32)],
    )(Fb, Fnb, centerInit)
    cout, labels, oh0, oh1, weight, cini = outs
    onehot = jnp.stack([oh0, oh1], axis=-1)
    return (cout, labels, onehot, weight, cini[0, 0])


# speedup vs baseline: 6.9023x; 6.9023x over previous
import jax
import jax.numpy as jnp
from jax.experimental import pallas as pl
from jax.experimental.pallas import tpu as pltpu

_N = 9216
_D = 384
_NB = 4
_TH = 0.01
_NG = _N // 8
_GPC = 165


def _rownorm(c):
    n2 = jnp.sum(c * c, axis=1, keepdims=True)
    return c / jnp.maximum(jnp.sqrt(n2), 1e-12)


def _tree8(acc):
    b = jnp.concatenate([acc[4:8], acc[0:4]], axis=0) + acc
    c = jnp.concatenate([b[2:8], b[0:2]], axis=0) + b
    d = jnp.concatenate([c[1:8], c[0:1]], axis=0) + c
    return d[0:1]


_UN = 15


def _masked_sums(fc_ref, m_ref):
    S0 = jnp.zeros((1, _D), jnp.float32)
    S1 = jnp.zeros((1, _D), jnp.float32)

    def group(base, a0, a1):
        f8 = fc_ref[pl.ds(base, 8), :]
        m8 = m_ref[pl.ds(base, 8), :]
        return f8 * (1.0 - m8) + a0, f8 * m8 + a1

    for c in range(-(-_NG // _GPC)):
        glo = c * _GPC
        gn = min(_GPC, _NG - glo)
        ntrip, rem = divmod(gn, _UN)

        def body(j, accs):
            a0, a1 = accs
            for u in range(_UN):
                a0, a1 = group((glo + j * _UN + u) * 8, a0, a1)
            return (a0, a1)

        a0, a1 = jax.lax.fori_loop(
            0, ntrip, body,
            (jnp.zeros((8, _D), jnp.float32), jnp.zeros((8, _D), jnp.float32)))
        for u in range(rem):
            a0, a1 = group((glo + ntrip * _UN + u) * 8, a0, a1)
        S0 = S0 + _tree8(a0)
        S1 = S1 + _tree8(a1)
    return jnp.concatenate([S0, S1], axis=0)


def _body(fc_ref, fnb_ref, c_ref, cout_ref, lab_ref, oh0_ref, oh1_ref,
          w_ref, cini_ref, m_ref):
    Fnb = fnb_ref[...]

    def step(centers):
        nc = _rownorm(centers)
        ncb = nc.astype(jnp.bfloat16)
        s = jax.lax.dot_general(
            ncb, Fnb, (((1,), (0,)), ((), ())),
            preferred_element_type=jnp.float32)
        d = 0.5 * (1.0 - s)
        lab = (d[1:2] < d[0:1]).astype(jnp.float32)
        m_ref[...] = lab.T
        S = _masked_sums(fc_ref, m_ref)
        cnt1 = jnp.sum(lab)
        cnts = jnp.concatenate(
            [jnp.reshape(_N - cnt1 + 1.0, (1, 1)),
             jnp.reshape(cnt1 + 1.0, (1, 1))], axis=0)
        cI = S / cnts
        Cd = jnp.sum(_rownorm(cI) * nc) * 0.5
        return d, lab, cI, Cd

    def cond_step(state):
        d, lab, cI, cdist, done = state
        d2, lb2, cI2, Cd2 = step(cI)
        d = jnp.where(done, d, d2)
        lab = jnp.where(done, lab, lb2)
        cI = jnp.where(done, cI, cI2)
        cdist = jnp.where(done, cdist, Cd2)
        done = jnp.logical_or(done, cdist < _TH)
        return (d, lab, cI, cdist, done), Cd2

    def record(b, state):
        d, lab, cI, _, _ = state
        lab_ref[b:b + 1, :] = lab.astype(jnp.int32)
        oh0_ref[b:b + 1, :] = 1.0 - lab
        oh1_ref[b:b + 1, :] = lab
        dmax = jnp.max(d, axis=1, keepdims=True)
        dmin = jnp.min(d, axis=1, keepdims=True)
        dn = -d / (dmax - dmin + 1e-7)
        w = 1.0 - dn + 0.1
        w_ref[b:b + 1, :] = w[0:1] * (1.0 - lab) + w[1:2] * lab

    d, lab, cI, Cd = step(c_ref[...])
    cini = Cd
    state = (d, lab, cI, Cd, Cd < _TH)
    for _ in range(3):
        state, _ = cond_step(state)
    record(0, state)
    csum = state[2]
    for b in range(1, _NB):
        done_prev = state[4]
        state, Cd2 = cond_step(state)
        cini = cini + jnp.where(done_prev, 0.0, Cd2)
        for _ in range(3):
            state, _ = cond_step(state)
        record(b, state)
        csum = csum + state[2]
    cout_ref[...] = csum / _NB
    cini_ref[...] = jnp.reshape(cini / _NB, (1, 1))


def kernel(FeatureT, centerInit):
    Fb = FeatureT[0].reshape(_D, _N).T
    n = jnp.linalg.norm(Fb, axis=1, keepdims=True)
    Fnb = (Fb / jnp.maximum(n, 1e-12)).astype(jnp.bfloat16).T
    outs = pl.pallas_call(
        _body,
        out_shape=(
            jax.ShapeDtypeStruct((2, _D), jnp.float32),
            jax.ShapeDtypeStruct((_NB, _N), jnp.int32),
            jax.ShapeDtypeStruct((_NB, _N), jnp.float32),
            jax.ShapeDtypeStruct((_NB, _N), jnp.float32),
            jax.ShapeDtypeStruct((_NB, _N), jnp.float32),
            jax.ShapeDtypeStruct((1, 1), jnp.float32),
        ),
        scratch_shapes=[pltpu.VMEM((_N, 1), jnp.float32)],
    )(Fb, Fnb, centerInit)
    cout, labels, oh0, oh1, weight, cini = outs
    onehot = jnp.stack([oh0, oh1], axis=-1)
    return (cout, labels, onehot, weight, cini[0, 0])


# speedup vs baseline: 8.8369x; 1.2803x over previous
import jax
import jax.numpy as jnp
from jax.experimental import pallas as pl
from jax.experimental.pallas import tpu as pltpu

_N = 9216
_D = 384
_NB = 4
_TH = 0.01
_NG = _N // 8
_GPC = 165


def _rownorm(c):
    n2 = jnp.sum(c * c, axis=1, keepdims=True)
    return c / jnp.maximum(jnp.sqrt(n2), 1e-12)


def _tree8(acc):
    b = jnp.concatenate([acc[4:8], acc[0:4]], axis=0) + acc
    c = jnp.concatenate([b[2:8], b[0:2]], axis=0) + b
    d = jnp.concatenate([c[1:8], c[0:1]], axis=0) + c
    return d[0:1]


_UN = 33


def _masked_sums(fc_ref, m_ref):
    S0 = jnp.zeros((1, _D), jnp.float32)
    S1 = jnp.zeros((1, _D), jnp.float32)

    def group(base, a0, a1):
        f8 = fc_ref[pl.ds(base, 8), :]
        m8 = m_ref[pl.ds(base, 8), :]
        return f8 * (1.0 - m8) + a0, f8 * m8 + a1

    for c in range(-(-_NG // _GPC)):
        glo = c * _GPC
        gn = min(_GPC, _NG - glo)
        ntrip, rem = divmod(gn, _UN)

        def body(j, accs):
            a0, a1 = accs
            for u in range(_UN):
                a0, a1 = group((glo + j * _UN + u) * 8, a0, a1)
            return (a0, a1)

        a0, a1 = jax.lax.fori_loop(
            0, ntrip, body,
            (jnp.zeros((8, _D), jnp.float32), jnp.zeros((8, _D), jnp.float32)))
        for u in range(rem):
            a0, a1 = group((glo + ntrip * _UN + u) * 8, a0, a1)
        S0 = S0 + _tree8(a0)
        S1 = S1 + _tree8(a1)
    return jnp.concatenate([S0, S1], axis=0)


def _body(fc_ref, fnb_ref, c_ref, cout_ref, lab_ref, oh0_ref, oh1_ref,
          w_ref, cini_ref, m_ref):
    Fnb = fnb_ref[...]

    def step(centers):
        nc = _rownorm(centers)
        ncb = nc.astype(jnp.bfloat16)
        s = jax.lax.dot_general(
            ncb, Fnb, (((1,), (0,)), ((), ())),
            preferred_element_type=jnp.float32)
        d = 0.5 * (1.0 - s)
        lab = (d[1:2] < d[0:1]).astype(jnp.float32)
        m_ref[...] = lab.T
        S = _masked_sums(fc_ref, m_ref)
        cnt1 = jnp.sum(lab)
        cnts = jnp.concatenate(
            [jnp.reshape(_N - cnt1 + 1.0, (1, 1)),
             jnp.reshape(cnt1 + 1.0, (1, 1))], axis=0)
        cI = S / cnts
        Cd = jnp.sum(_rownorm(cI) * nc) * 0.5
        return d, lab, cI, Cd

    def cond_step(state):
        d, lab, cI, cdist, done = state
        d2, lb2, cI2, Cd2 = step(cI)
        d = jnp.where(done, d, d2)
        lab = jnp.where(done, lab, lb2)
        cI = jnp.where(done, cI, cI2)
        cdist = jnp.where(done, cdist, Cd2)
        done = jnp.logical_or(done, cdist < _TH)
        return (d, lab, cI, cdist, done), Cd2

    def record(b, state):
        d, lab, cI, _, _ = state
        lab_ref[b:b + 1, :] = lab.astype(jnp.int32)
        oh0_ref[b:b + 1, :] = 1.0 - lab
        oh1_ref[b:b + 1, :] = lab
        dmax = jnp.max(d, axis=1, keepdims=True)
        dmin = jnp.min(d, axis=1, keepdims=True)
        dn = -d / (dmax - dmin + 1e-7)
        w = 1.0 - dn + 0.1
        w_ref[b:b + 1, :] = w[0:1] * (1.0 - lab) + w[1:2] * lab

    d, lab, cI, Cd = step(c_ref[...])
    cini = Cd
    state = (d, lab, cI, Cd, Cd < _TH)
    for _ in range(3):
        state, _ = cond_step(state)
    record(0, state)
    csum = state[2]
    for b in range(1, _NB):
        done_prev = state[4]
        state, Cd2 = cond_step(state)
        cini = cini + jnp.where(done_prev, 0.0, Cd2)
        for _ in range(3):
            state, _ = cond_step(state)
        record(b, state)
        csum = csum + state[2]
    cout_ref[...] = csum / _NB
    cini_ref[...] = jnp.reshape(cini / _NB, (1, 1))


def kernel(FeatureT, centerInit):
    Fb = FeatureT[0].reshape(_D, _N).T
    n = jnp.linalg.norm(Fb, axis=1, keepdims=True)
    Fnb = (Fb / jnp.maximum(n, 1e-12)).astype(jnp.bfloat16).T
    outs = pl.pallas_call(
        _body,
        out_shape=(
            jax.ShapeDtypeStruct((2, _D), jnp.float32),
            jax.ShapeDtypeStruct((_NB, _N), jnp.int32),
            jax.ShapeDtypeStruct((_NB, _N), jnp.float32),
            jax.ShapeDtypeStruct((_NB, _N), jnp.float32),
            jax.ShapeDtypeStruct((_NB, _N), jnp.float32),
            jax.ShapeDtypeStruct((1, 1), jnp.float32),
        ),
        scratch_shapes=[pltpu.VMEM((_N, 1), jnp.float32)],
    )(Fb, Fnb, centerInit)
    cout, labels, oh0, oh1, weight, cini = outs
    onehot = jnp.stack([oh0, oh1], axis=-1)
    return (cout, labels, onehot, weight, cini[0, 0])


# speedup vs baseline: 11.2680x; 1.2751x over previous
import jax
import jax.numpy as jnp
from jax.experimental import pallas as pl
from jax.experimental.pallas import tpu as pltpu

_N = 9216
_D = 384
_NB = 4
_TH = 0.01
_NG = _N // 8
_GPC = 165


def _rownorm(c):
    n2 = jnp.sum(c * c, axis=1, keepdims=True)
    return c / jnp.maximum(jnp.sqrt(n2), 1e-12)


def _tree8(acc):
    b = jnp.concatenate([acc[4:8], acc[0:4]], axis=0) + acc
    c = jnp.concatenate([b[2:8], b[0:2]], axis=0) + b
    d = jnp.concatenate([c[1:8], c[0:1]], axis=0) + c
    return d[0:1]


_UN = 165


def _masked_sums(fc_ref, m_ref):
    S0 = jnp.zeros((1, _D), jnp.float32)
    S1 = jnp.zeros((1, _D), jnp.float32)

    def group(base, a0, a1):
        f8 = fc_ref[pl.ds(base, 8), :]
        m8 = m_ref[pl.ds(base, 8), :]
        return f8 * (1.0 - m8) + a0, f8 * m8 + a1

    for c in range(-(-_NG // _GPC)):
        glo = c * _GPC
        gn = min(_GPC, _NG - glo)
        ntrip, rem = divmod(gn, _UN)
        a0 = jnp.zeros((8, _D), jnp.float32)
        a1 = jnp.zeros((8, _D), jnp.float32)
        if ntrip > 1:
            def body(j, accs):
                b0, b1 = accs
                for u in range(_UN):
                    b0, b1 = group((glo + j * _UN + u) * 8, b0, b1)
                return (b0, b1)
            a0, a1 = jax.lax.fori_loop(0, ntrip, body, (a0, a1))
            done_g = ntrip * _UN
        else:
            done_g = 0
            rem = gn
        for u in range(rem):
            a0, a1 = group((glo + done_g + u) * 8, a0, a1)
        S0 = S0 + _tree8(a0)
        S1 = S1 + _tree8(a1)
    return jnp.concatenate([S0, S1], axis=0)


def _body(fc_ref, fnb_ref, c_ref, cout_ref, lab_ref, oh0_ref, oh1_ref,
          w_ref, cini_ref, m_ref):
    Fnb = fnb_ref[...]

    def step(centers):
        nc = _rownorm(centers)
        ncb = nc.astype(jnp.bfloat16)
        s = jax.lax.dot_general(
            ncb, Fnb, (((1,), (0,)), ((), ())),
            preferred_element_type=jnp.float32)
        d = 0.5 * (1.0 - s)
        lab = (d[1:2] < d[0:1]).astype(jnp.float32)
        m_ref[...] = lab.T
        S = _masked_sums(fc_ref, m_ref)
        cnt1 = jnp.sum(lab)
        cnts = jnp.concatenate(
            [jnp.reshape(_N - cnt1 + 1.0, (1, 1)),
             jnp.reshape(cnt1 + 1.0, (1, 1))], axis=0)
        cI = S / cnts
        Cd = jnp.sum(_rownorm(cI) * nc) * 0.5
        return d, lab, cI, Cd

    def cond_step(state):
        d, lab, cI, cdist, done = state
        d2, lb2, cI2, Cd2 = step(cI)
        d = jnp.where(done, d, d2)
        lab = jnp.where(done, lab, lb2)
        cI = jnp.where(done, cI, cI2)
        cdist = jnp.where(done, cdist, Cd2)
        done = jnp.logical_or(done, cdist < _TH)
        return (d, lab, cI, cdist, done), Cd2

    def record(b, state):
        d, lab, cI, _, _ = state
        lab_ref[b:b + 1, :] = lab.astype(jnp.int32)
        oh0_ref[b:b + 1, :] = 1.0 - lab
        oh1_ref[b:b + 1, :] = lab
        dmax = jnp.max(d, axis=1, keepdims=True)
        dmin = jnp.min(d, axis=1, keepdims=True)
        dn = -d / (dmax - dmin + 1e-7)
        w = 1.0 - dn + 0.1
        w_ref[b:b + 1, :] = w[0:1] * (1.0 - lab) + w[1:2] * lab

    d, lab, cI, Cd = step(c_ref[...])
    cini = Cd
    state = (d, lab, cI, Cd, Cd < _TH)
    for _ in range(3):
        state, _ = cond_step(state)
    record(0, state)
    csum = state[2]
    for b in range(1, _NB):
        done_prev = state[4]
        state, Cd2 = cond_step(state)
        cini = cini + jnp.where(done_prev, 0.0, Cd2)
        for _ in range(3):
            state, _ = cond_step(state)
        record(b, state)
        csum = csum + state[2]
    cout_ref[...] = csum / _NB
    cini_ref[...] = jnp.reshape(cini / _NB, (1, 1))


def kernel(FeatureT, centerInit):
    Fb = FeatureT[0].reshape(_D, _N).T
    n = jnp.linalg.norm(Fb, axis=1, keepdims=True)
    Fnb = (Fb / jnp.maximum(n, 1e-12)).astype(jnp.bfloat16).T
    outs = pl.pallas_call(
        _body,
        out_shape=(
            jax.ShapeDtypeStruct((2, _D), jnp.float32),
            jax.ShapeDtypeStruct((_NB, _N), jnp.int32),
            jax.ShapeDtypeStruct((_NB, _N), jnp.float32),
            jax.ShapeDtypeStruct((_NB, _N), jnp.float32),
            jax.ShapeDtypeStruct((_NB, _N), jnp.float32),
            jax.ShapeDtypeStruct((1, 1), jnp.float32),
        ),
        scratch_shapes=[pltpu.VMEM((_N, 1), jnp.float32)],
    )(Fb, Fnb, centerInit)
    cout, labels, oh0, oh1, weight, cini = outs
    onehot = jnp.stack([oh0, oh1], axis=-1)
    return (cout, labels, onehot, weight, cini[0, 0])


# speedup vs baseline: 12.3583x; 1.0968x over previous
import jax
import jax.numpy as jnp
from jax.experimental import pallas as pl
from jax.experimental.pallas import tpu as pltpu

_N = 9216
_D = 384
_NB = 4
_TH = 0.01
_NG = _N // 8
_GPC = 165


def _rownorm(c):
    n2 = jnp.sum(c * c, axis=1, keepdims=True)
    return c / jnp.maximum(jnp.sqrt(n2), 1e-12)


def _tree8(acc):
    b = jnp.concatenate([acc[4:8], acc[0:4]], axis=0) + acc
    c = jnp.concatenate([b[2:8], b[0:2]], axis=0) + b
    d = jnp.concatenate([c[1:8], c[0:1]], axis=0) + c
    return d[0:1]


_UN = 165


def _masked_sums(fc_ref, m_ref):
    S0 = jnp.zeros((1, _D), jnp.float32)
    S1 = jnp.zeros((1, _D), jnp.float32)

    def group(base, a0, a1):
        f8 = fc_ref[pl.ds(base, 8), :]
        m8 = m_ref[pl.ds(base, 8), :]
        p1 = f8 * m8
        return (f8 - p1) + a0, p1 + a1

    for c in range(-(-_NG // _GPC)):
        glo = c * _GPC
        gn = min(_GPC, _NG - glo)
        ntrip, rem = divmod(gn, _UN)
        a0 = jnp.zeros((8, _D), jnp.float32)
        a1 = jnp.zeros((8, _D), jnp.float32)
        if ntrip > 1:
            def body(j, accs):
                b0, b1 = accs
                for u in range(_UN):
                    b0, b1 = group((glo + j * _UN + u) * 8, b0, b1)
                return (b0, b1)
            a0, a1 = jax.lax.fori_loop(0, ntrip, body, (a0, a1))
            done_g = ntrip * _UN
        else:
            done_g = 0
            rem = gn
        for u in range(rem):
            a0, a1 = group((glo + done_g + u) * 8, a0, a1)
        S0 = S0 + _tree8(a0)
        S1 = S1 + _tree8(a1)
    return jnp.concatenate([S0, S1], axis=0)


def _body(fc_ref, fnb_ref, c_ref, cout_ref, lab_ref, oh0_ref, oh1_ref,
          w_ref, cini_ref, m_ref):
    Fnb = fnb_ref[...]

    def step(centers):
        nc = _rownorm(centers)
        ncb = nc.astype(jnp.bfloat16)
        s = jax.lax.dot_general(
            ncb, Fnb, (((1,), (0,)), ((), ())),
            preferred_element_type=jnp.float32)
        d = 0.5 * (1.0 - s)
        lab = (d[1:2] < d[0:1]).astype(jnp.float32)
        m_ref[...] = lab.T
        S = _masked_sums(fc_ref, m_ref)
        cnt1 = jnp.sum(lab)
        cnts = jnp.concatenate(
            [jnp.reshape(_N - cnt1 + 1.0, (1, 1)),
             jnp.reshape(cnt1 + 1.0, (1, 1))], axis=0)
        cI = S / cnts
        Cd = jnp.sum(_rownorm(cI) * nc) * 0.5
        return d, lab, cI, Cd

    def cond_step(state):
        d, lab, cI, cdist, done = state
        d2, lb2, cI2, Cd2 = step(cI)
        d = jnp.where(done, d, d2)
        lab = jnp.where(done, lab, lb2)
        cI = jnp.where(done, cI, cI2)
        cdist = jnp.where(done, cdist, Cd2)
        done = jnp.logical_or(done, cdist < _TH)
        return (d, lab, cI, cdist, done), Cd2

    def record(b, state):
        d, lab, cI, _, _ = state
        lab_ref[b:b + 1, :] = lab.astype(jnp.int32)
        oh0_ref[b:b + 1, :] = 1.0 - lab
        oh1_ref[b:b + 1, :] = lab
        dmax = jnp.max(d, axis=1, keepdims=True)
        dmin = jnp.min(d, axis=1, keepdims=True)
        dn = -d / (dmax - dmin + 1e-7)
        w = 1.0 - dn + 0.1
        w_ref[b:b + 1, :] = w[0:1] * (1.0 - lab) + w[1:2] * lab

    d, lab, cI, Cd = step(c_ref[...])
    cini = Cd
    state = (d, lab, cI, Cd, Cd < _TH)
    for _ in range(3):
        state, _ = cond_step(state)
    record(0, state)
    csum = state[2]
    for b in range(1, _NB):
        done_prev = state[4]
        state, Cd2 = cond_step(state)
        cini = cini + jnp.where(done_prev, 0.0, Cd2)
        for _ in range(3):
            state, _ = cond_step(state)
        record(b, state)
        csum = csum + state[2]
    cout_ref[...] = csum / _NB
    cini_ref[...] = jnp.reshape(cini / _NB, (1, 1))


def kernel(FeatureT, centerInit):
    Fb = FeatureT[0].reshape(_D, _N).T
    n = jnp.linalg.norm(Fb, axis=1, keepdims=True)
    Fnb = (Fb / jnp.maximum(n, 1e-12)).astype(jnp.bfloat16).T
    outs = pl.pallas_call(
        _body,
        out_shape=(
            jax.ShapeDtypeStruct((2, _D), jnp.float32),
            jax.ShapeDtypeStruct((_NB, _N), jnp.int32),
            jax.ShapeDtypeStruct((_NB, _N), jnp.float32),
            jax.ShapeDtypeStruct((_NB, _N), jnp.float32),
            jax.ShapeDtypeStruct((_NB, _N), jnp.float32),
            jax.ShapeDtypeStruct((1, 1), jnp.float32),
        ),
        scratch_shapes=[pltpu.VMEM((_N, 1), jnp.float32)],
    )(Fb, Fnb, centerInit)
    cout, labels, oh0, oh1, weight, cini = outs
    onehot = jnp.stack([oh0, oh1], axis=-1)
    return (cout, labels, onehot, weight, cini[0, 0])
